# Initial kernel scaffold; baseline (speedup 1.0000x reference)
import jax
import jax.numpy as jnp
from jax.experimental import pallas as pl


def kernel(tokens, weights, fc_w, fc_b):
    raise NotImplementedError("write your pallas kernel here")


# speedup vs baseline: 1.9512x; 1.9512x over previous
import functools

import jax
import jax.numpy as jnp
from jax import lax
from jax.experimental import pallas as pl
from jax.experimental.pallas import tpu as pltpu
from jax.experimental.pallas import tpu_sc as plsc

_VOCAB = 1000
_D_EMB = 128
_D_MODEL = 512

_NC = 2
_NS = 16
_NW = _NC * _NS


def _fuse_body(w_ref, fw_ref, b_ref, o_ref):
    o_ref[...] = (
        jnp.dot(w_ref[...], fw_ref[...], preferred_element_type=jnp.float32)
        + b_ref[...]
    )


def _fused_table(weights, fc_w, fc_b):
    return pl.pallas_call(
        _fuse_body,
        out_shape=jax.ShapeDtypeStruct((_VOCAB, _D_MODEL), jnp.float32),
    )(weights, fc_w, fc_b.reshape(1, _D_MODEL))


def _make_gather(n_tokens, d):
    b_per_w = n_tokens // _NW
    ch = 64
    n_chunks = b_per_w // ch
    assert n_tokens % _NW == 0 and b_per_w % ch == 0

    mesh = plsc.VectorSubcoreMesh(core_axis_name="c", subcore_axis_name="s")

    @functools.partial(
        pl.kernel,
        mesh=mesh,
        out_type=jax.ShapeDtypeStruct((n_tokens, d), jnp.float32),
        scratch_types=[
            pltpu.VMEM((n_chunks, ch), jnp.int32),
            pltpu.VMEM((ch, d), jnp.float32),
            pltpu.SemaphoreType.DMA,
        ],
    )
    def gather(table_hbm, tok_hbm, out_hbm, idx_v, rows_v, sem):
        wid = lax.axis_index("s") * _NC + lax.axis_index("c")
        base = wid * b_per_w
        pltpu.sync_copy(tok_hbm.at[wid], idx_v)

        def chunk(c, carry):
            pltpu.async_copy(table_hbm.at[idx_v.at[c]], rows_v, sem).wait()
            pltpu.sync_copy(rows_v, out_hbm.at[pl.ds(base + c * ch, ch)])
            return carry

        lax.fori_loop(0, n_chunks, chunk, 0)

    def run(table, tokens_flat):
        toks = tokens_flat.reshape(_NW, n_chunks, ch)
        return gather(table, toks)

    return run


def kernel(tokens, weights, fc_w, fc_b):
    bs, seq = tokens.shape
    fused = _fused_table(weights, fc_w, fc_b)
    tok_flat = tokens.reshape(-1).astype(jnp.int32)
    out = _make_gather(bs * seq, _D_MODEL)(fused, tok_flat)
    return out.reshape(bs, seq, _D_MODEL)


# speedup vs baseline: 2.0250x; 1.0378x over previous
import functools

import jax
import jax.numpy as jnp
from jax import lax
from jax.experimental import pallas as pl
from jax.experimental.pallas import tpu as pltpu
from jax.experimental.pallas import tpu_sc as plsc

_VOCAB = 1000
_D_EMB = 128
_D_MODEL = 512

_NC = 2
_NS = 16
_NW = _NC * _NS


def _fuse_body(w_ref, fw_ref, b_ref, o_ref):
    o_ref[...] = (
        jnp.dot(w_ref[...], fw_ref[...], preferred_element_type=jnp.float32)
        + b_ref[...]
    )


def _fused_table(weights, fc_w, fc_b):
    return pl.pallas_call(
        _fuse_body,
        out_shape=jax.ShapeDtypeStruct((_VOCAB, _D_MODEL), jnp.float32),
    )(weights, fc_w, fc_b.reshape(1, _D_MODEL))


def _make_gather(n_tokens, d, ch=40, nbuf=4):
    b_per_w = n_tokens // _NW
    n_chunks = b_per_w // ch
    lookahead = nbuf // 2
    assert n_tokens % _NW == 0 and b_per_w % ch == 0 and n_chunks % nbuf == 0

    mesh = plsc.VectorSubcoreMesh(core_axis_name="c", subcore_axis_name="s")

    @functools.partial(
        pl.kernel,
        mesh=mesh,
        out_type=jax.ShapeDtypeStruct((n_tokens, d), jnp.float32),
        scratch_types=[
            pltpu.VMEM((n_chunks, ch), jnp.int32),
            pltpu.VMEM((nbuf, ch, d), jnp.float32),
        ]
        + [pltpu.SemaphoreType.DMA] * (2 * nbuf),
    )
    def gather(table_hbm, tok_hbm, out_hbm, idx_v, rows_v, *sems):
        gsem, wsem = sems[:nbuf], sems[nbuf:]
        wid = lax.axis_index("s") * _NC + lax.axis_index("c")
        base = wid * b_per_w
        pltpu.sync_copy(tok_hbm.at[wid], idx_v)

        def g_copy(c, b):
            return pltpu.make_async_copy(
                table_hbm.at[idx_v.at[c]], rows_v.at[b], gsem[b]
            )

        def w_copy(c, b):
            return pltpu.make_async_copy(
                rows_v.at[b], out_hbm.at[pl.ds(base + c * ch, ch)], wsem[b]
            )

        for s in range(lookahead):
            g_copy(s, s % nbuf).start()

        def outer(i, carry):
            for j in range(nbuf):
                s = i * nbuf + j
                bg = (j + lookahead) % nbuf

                @pl.when(
                    (s + lookahead - nbuf >= 0) & (s + lookahead < n_chunks)
                )
                def _():
                    w_copy(0, bg).wait()

                @pl.when(s + lookahead < n_chunks)
                def _():
                    g_copy(s + lookahead, bg).start()

                g_copy(0, j).wait()
                w_copy(s, j).start()
            return carry

        lax.fori_loop(0, n_chunks // nbuf, outer, 0)
        for j in range(nbuf):
            w_copy(0, j).wait()

    def run(table, tokens_flat):
        toks = tokens_flat.reshape(_NW, n_chunks, ch)
        return gather(table, toks)

    return run


def kernel(tokens, weights, fc_w, fc_b):
    bs, seq = tokens.shape
    fused = _fused_table(weights, fc_w, fc_b)
    tok_flat = tokens.reshape(-1).astype(jnp.int32)
    out = _make_gather(bs * seq, _D_MODEL)(fused, tok_flat)
    return out.reshape(bs, seq, _D_MODEL)


# speedup vs baseline: 2.8847x; 1.4245x over previous
import functools

import jax
import jax.numpy as jnp
from jax import lax
from jax.experimental import pallas as pl
from jax.experimental.pallas import tpu as pltpu
from jax.experimental.pallas import tpu_sc as plsc

_VOCAB = 1000
_D_EMB = 128
_D_MODEL = 512

_NC = 2
_NS = 16
_NW = _NC * _NS


def _fuse_body(w_ref, fw_ref, b_ref, o_ref):
    o_ref[...] = (
        jnp.dot(w_ref[...], fw_ref[...], preferred_element_type=jnp.float32)
        + b_ref[...]
    )


def _fused_table(weights, fc_w, fc_b):
    return pl.pallas_call(
        _fuse_body,
        out_shape=jax.ShapeDtypeStruct((_VOCAB, _D_MODEL), jnp.float32),
    )(weights, fc_w, fc_b.reshape(1, _D_MODEL))


def _make_gather(bs, seq, d, nbuf=4):
    n_chunks = bs // _NW
    assert bs % _NW == 0 and n_chunks % nbuf == 0 and n_chunks >= nbuf

    mesh = plsc.VectorSubcoreMesh(core_axis_name="c", subcore_axis_name="s")

    @functools.partial(
        pl.kernel,
        mesh=mesh,
        out_type=jax.ShapeDtypeStruct((bs, seq, d), jnp.float32),
        scratch_types=[pltpu.VMEM((seq,), jnp.int32)] * nbuf
        + [pltpu.VMEM((seq, d), jnp.float32)] * nbuf
        + [pltpu.SemaphoreType.DMA] * (3 * nbuf),
    )
    def gather(table_hbm, tok_hbm, out_hbm, *scratch):
        idx_v = scratch[:nbuf]
        rows_v = scratch[nbuf : 2 * nbuf]
        isem = scratch[2 * nbuf : 3 * nbuf]
        gsem = scratch[3 * nbuf : 4 * nbuf]
        wsem = scratch[4 * nbuf :]
        wid = lax.axis_index("s") * _NC + lax.axis_index("c")
        base = wid * n_chunks

        def i_copy(c, b):
            return pltpu.make_async_copy(tok_hbm.at[base + c], idx_v[b], isem[b])

        def g_copy(c, b):
            return pltpu.make_async_copy(
                table_hbm.at[idx_v[b]], rows_v[b], gsem[b]
            )

        def w_copy(c, b):
            return pltpu.make_async_copy(
                rows_v[b], out_hbm.at[base + c], wsem[b]
            )

        for c in range(3):
            i_copy(c, c % nbuf).start()
        for c in range(2):
            i_copy(0, c % nbuf).wait()
            g_copy(c, c % nbuf).start()

        def outer(i, carry):
            for j in range(nbuf):
                s = i * nbuf + j
                b2 = (j + 2) % nbuf

                @pl.when(s + 3 < n_chunks)
                def _():
                    i_copy(s + 3, (j + 3) % nbuf).start()

                @pl.when((s + 2 < n_chunks) & (s >= 2))
                def _():
                    w_copy(0, b2).wait()

                @pl.when(s + 2 < n_chunks)
                def _():
                    i_copy(0, b2).wait()
                    g_copy(s + 2, b2).start()

                g_copy(0, j).wait()
                w_copy(s, j).start()
            return carry

        lax.fori_loop(0, n_chunks // nbuf, outer, 0)
        for j in range(nbuf):
            w_copy(0, j).wait()

    return lambda table, tokens: gather(table, tokens)


def kernel(tokens, weights, fc_w, fc_b):
    bs, seq = tokens.shape
    fused = _fused_table(weights, fc_w, fc_b)
    return _make_gather(bs, seq, _D_MODEL)(fused, tokens.astype(jnp.int32))


# speedup vs baseline: 5.4262x; 1.8811x over previous
import functools

import jax
import jax.numpy as jnp
from jax import lax
from jax.experimental import pallas as pl
from jax.experimental.pallas import tpu as pltpu
from jax.experimental.pallas import tpu_sc as plsc

_VOCAB = 1000
_D_EMB = 128
_D_MODEL = 512

_NC = 2
_NS = 16
_NW = _NC * _NS


def _fuse_body(w_ref, fw_ref, b_ref, o_ref):
    o_ref[...] = (
        jnp.dot(w_ref[...], fw_ref[...], preferred_element_type=jnp.float32)
        + b_ref[...]
    )


def _fused_table(weights, fc_w, fc_b):
    return pl.pallas_call(
        _fuse_body,
        out_shape=jax.ShapeDtypeStruct((_VOCAB, _D_MODEL), jnp.float32),
    )(weights, fc_w, fc_b.reshape(1, _D_MODEL))


def _make_gather(bs, seq, d, ch=32, nbuf=5):
    n_tok = bs * seq
    per_w = n_tok // _NW
    n_chunks = per_w // ch
    chunks_per_plane = bs // ch
    assert n_tok % _NW == 0 and per_w % ch == 0 and n_chunks % nbuf == 0
    assert bs % ch == 0 and ch % 8 == 0

    mesh = plsc.VectorSubcoreMesh(core_axis_name="c", subcore_axis_name="s")

    @functools.partial(
        pl.kernel,
        mesh=mesh,
        out_type=jax.ShapeDtypeStruct((seq, bs, d), jnp.float32),
        scratch_types=[
            pltpu.VMEM((n_chunks, ch), jnp.int32),
            pltpu.VMEM((nbuf, ch, d), jnp.float32),
        ]
        + [pltpu.SemaphoreType.DMA] * (2 * nbuf),
    )
    def gather(table_hbm, tok_hbm, out_hbm, idx_v, rows_v, *sems):
        gsem, wsem = sems[:nbuf], sems[nbuf:]
        wid = lax.axis_index("s") * _NC + lax.axis_index("c")
        gbase = wid * n_chunks
        pltpu.sync_copy(tok_hbm.at[wid], idx_v)

        def g_copy(c, b):
            return pltpu.make_async_copy(
                table_hbm.at[idx_v.at[c]], rows_v.at[b], gsem[b]
            )

        def w_copy(c, b):
            g = gbase + c
            sp = g // chunks_per_plane
            b0 = (g % chunks_per_plane) * ch
            return pltpu.make_async_copy(
                rows_v.at[b], out_hbm.at[sp, pl.ds(b0, ch)], wsem[b]
            )

        for c in range(2):
            g_copy(c, c % nbuf).start()

        def outer(i, carry):
            for j in range(nbuf):
                s = i * nbuf + j
                bg = (j + 2) % nbuf

                @pl.when((s + 2 < n_chunks) & (s - 3 >= 0))
                def _():
                    w_copy(0, bg).wait()

                @pl.when(s + 2 < n_chunks)
                def _():
                    g_copy(s + 2, bg).start()

                g_copy(0, j).wait()
                w_copy(s, j).start()
            return carry

        lax.fori_loop(0, n_chunks // nbuf, outer, 0)
        for j in range(nbuf):
            w_copy(0, j).wait()

    def run(table, tokens):
        toks = tokens.T.reshape(_NW, n_chunks, ch)
        out = gather(table, toks)
        return out.transpose(1, 0, 2)

    return run


def kernel(tokens, weights, fc_w, fc_b):
    bs, seq = tokens.shape
    fused = _fused_table(weights, fc_w, fc_b)
    return _make_gather(bs, seq, _D_MODEL)(fused, tokens.astype(jnp.int32))
